# baseline (speedup 1.0000x reference)
import jax
import jax.numpy as jnp
from jax import lax
from jax.experimental import pallas as pl
from jax.experimental.pallas import tpu as pltpu
from jax.experimental.pallas import tpu_sc as plsc

N_NODES = 10000
D = 128
E = 320000
N_PAD = 10240
NC, NS, L = 2, 16, 16
NW = NC * NS
E_PER_W = E // NW
CH = 128
E_PAD_PER_W = 10240
NCHUNK = E_PAD_PER_W // CH
DEG_ROWS = N_PAD // D
ACC_ROWS_PER_TILE = N_PAD // NS
WB_CH = 64
RB = 1024

_mesh = plsc.VectorSubcoreMesh(core_axis_name="c", subcore_axis_name="s")
_sc_params = pltpu.CompilerParams(needs_layout_passes=False)


def _deg_body(dst_hbm, out_hbm, idx_v, deg_v):
    c = lax.axis_index("c")
    s = lax.axis_index("s")
    wid = c * NS + s

    pltpu.sync_copy(dst_hbm.at[wid], idx_v)

    zeros16 = jnp.zeros((L,), jnp.float32)

    def zero_row(i, carry):
        deg_v[pl.ds(i * L, L)] = zeros16
        return carry

    lax.fori_loop(0, N_PAD // L, zero_row, 0)

    ones16 = jnp.full((L,), 1.0, jnp.float32)

    def body(i, carry):
        vec = idx_v[pl.ds(i * L, L)]
        plsc.addupdate_scatter(deg_v, [vec], ones16)
        return carry

    lax.fori_loop(0, E_PER_W // L, body, 0)

    pltpu.sync_copy(deg_v, out_hbm.at[wid])


_deg_call = pl.kernel(
    _deg_body,
    out_type=jax.ShapeDtypeStruct((NW, N_PAD), jnp.float32),
    mesh=_mesh,
    scratch_types=[
        pltpu.VMEM((E_PER_W,), jnp.int32),
        pltpu.VMEM((N_PAD,), jnp.float32),
    ],
    compiler_params=_sc_params,
)


def _mm_body(x_ref, w_ref, h_ref):
    h_ref[...] = lax.dot_general(
        x_ref[...], w_ref[...], (((1,), (1,)), ((), ())),
        preferred_element_type=jnp.float32)


_mm_call = pl.pallas_call(
    _mm_body,
    grid=(N_PAD // RB,),
    in_specs=[
        pl.BlockSpec((RB, D), lambda i: (i, 0)),
        pl.BlockSpec((D, D), lambda i: (0, 0)),
    ],
    out_specs=pl.BlockSpec((RB, D), lambda i: (i, 0)),
    out_shape=jax.ShapeDtypeStruct((N_PAD, D), jnp.float32),
)


def _prep_body(h_ref, degp_ref, g_ref, dinv_ref):
    d = jnp.sum(degp_ref[...], axis=0) + 1.0
    dinv = lax.rsqrt(d)
    dinv_ref[...] = dinv
    g_ref[...] = h_ref[...] * dinv[:, None]


_prep_call = pl.pallas_call(
    _prep_body,
    grid=(N_PAD // RB,),
    in_specs=[
        pl.BlockSpec((RB, D), lambda i: (i, 0)),
        pl.BlockSpec((NW, RB), lambda i: (0, i)),
    ],
    out_specs=[
        pl.BlockSpec((RB, D), lambda i: (i, 0)),
        pl.BlockSpec((RB,), lambda i: (i,)),
    ],
    out_shape=[
        jax.ShapeDtypeStruct((N_PAD, D), jnp.float32),
        jax.ShapeDtypeStruct((N_PAD,), jnp.float32),
    ],
)


HNCHUNK = NCHUNK // 2


def _edge_body(src_hbm, dst_hbm, g_hbm, out_hbm, src_v, dst_v, rows0_v,
               rows1_v, acc_sh, semg0, semg1):
    c = lax.axis_index("c")
    s = lax.axis_index("s")
    wid = c * NS + s

    zeros16 = jnp.zeros((L,), jnp.float32)

    def zero_row(i, carry):
        for j in range(D // L):
            rows0_v[i, pl.ds(j * L, L)] = zeros16
        return carry

    lax.fori_loop(0, WB_CH, zero_row, 0)

    base = s * ACC_ROWS_PER_TILE
    zsrc = rows0_v.at[pl.ds(0, WB_CH)]

    def zero_acc(k, carry):
        pltpu.sync_copy(zsrc, acc_sh.at[pl.ds(base + k * WB_CH, WB_CH)])
        return carry

    lax.fori_loop(0, ACC_ROWS_PER_TILE // WB_CH, zero_acc, 0)

    plsc.subcore_barrier()

    for h in range(NCHUNK // HNCHUNK):
        pltpu.sync_copy(src_hbm.at[wid].at[pl.ds(h * HNCHUNK, HNCHUNK)],
                        src_v)
        pltpu.sync_copy(dst_hbm.at[wid].at[pl.ds(h * HNCHUNK, HNCHUNK)],
                        dst_v)
        pltpu.async_copy(g_hbm.at[src_v.at[0]], rows0_v, semg0)

        def body(i, carry):
            j = i * 2
            pltpu.async_copy(g_hbm.at[src_v.at[j + 1]], rows1_v, semg1)
            pltpu.make_async_copy(g_hbm.at[src_v.at[j]], rows0_v,
                                  semg0).wait()
            pltpu.sync_copy(rows0_v, acc_sh.at[dst_v.at[j]], add=True)

            @pl.when(j + 2 < HNCHUNK)
            def _():
                pltpu.async_copy(g_hbm.at[src_v.at[j + 2]], rows0_v, semg0)

            pltpu.make_async_copy(g_hbm.at[src_v.at[j + 1]], rows1_v,
                                  semg1).wait()
            pltpu.sync_copy(rows1_v, acc_sh.at[dst_v.at[j + 1]], add=True)
            return carry

        lax.fori_loop(0, HNCHUNK // 2, body, 0)

    plsc.subcore_barrier()

    wb = rows0_v.at[pl.ds(0, WB_CH)]

    def writeback(k, carry):
        r = base + k * WB_CH
        pltpu.sync_copy(acc_sh.at[pl.ds(r, WB_CH)], wb)
        pltpu.sync_copy(wb, out_hbm.at[c].at[pl.ds(r, WB_CH)])
        return carry

    lax.fori_loop(0, ACC_ROWS_PER_TILE // WB_CH, writeback, 0)


_edge_call = pl.kernel(
    _edge_body,
    out_type=jax.ShapeDtypeStruct((NC, N_PAD, D), jnp.float32),
    mesh=_mesh,
    scratch_types=[
        pltpu.VMEM((HNCHUNK, CH), jnp.int32),
        pltpu.VMEM((HNCHUNK, CH), jnp.int32),
        pltpu.VMEM((CH, D), jnp.float32),
        pltpu.VMEM((CH, D), jnp.float32),
        pltpu.VMEM_SHARED((N_PAD, D), jnp.float32),
        pltpu.SemaphoreType.DMA,
        pltpu.SemaphoreType.DMA,
    ],
    compiler_params=_sc_params,
)


def _fin_body(p_ref, g_ref, dinv_ref, b_ref, o_ref):
    acc = p_ref[0] + p_ref[1] + g_ref[...]
    o_ref[...] = jnp.maximum(acc * dinv_ref[...][:, None] + b_ref[...], 0.0)


_fin_call = pl.pallas_call(
    _fin_body,
    grid=(N_PAD // RB,),
    in_specs=[
        pl.BlockSpec((NC, RB, D), lambda i: (0, i, 0)),
        pl.BlockSpec((RB, D), lambda i: (i, 0)),
        pl.BlockSpec((RB,), lambda i: (i,)),
        pl.BlockSpec((1, D), lambda i: (0, 0)),
    ],
    out_specs=pl.BlockSpec((RB, D), lambda i: (i, 0)),
    out_shape=jax.ShapeDtypeStruct((N_PAD, D), jnp.float32),
)


def kernel(x, edge_index, pos, batch, W, b):
    src = edge_index[0].astype(jnp.int32)
    dst = edge_index[1].astype(jnp.int32)
    pad_w = E_PAD_PER_W - E_PER_W
    pad_ids = (jnp.arange(pad_w, dtype=jnp.int32) % (N_PAD - N_NODES)
               ) + N_NODES
    pad_blk = jnp.broadcast_to(pad_ids, (NW, pad_w))
    src_p = jnp.concatenate([src.reshape(NW, E_PER_W), pad_blk],
                            axis=1).reshape(NW, NCHUNK, CH)
    dst_p = jnp.concatenate([dst.reshape(NW, E_PER_W), pad_blk],
                            axis=1).reshape(NW, NCHUNK, CH)
    dst_flat = dst.reshape(NW, E_PER_W)

    xp = jnp.pad(x, ((0, N_PAD - N_NODES), (0, 0)))

    degp = _deg_call(dst_flat)
    h = _mm_call(xp, W)
    g, dinv = _prep_call(h, degp)
    p = _edge_call(src_p, dst_p, g)
    out_pad = _fin_call(p, g, dinv, b.reshape(1, D))
    return (out_pad[:N_NODES], edge_index, pos)


# speedup vs baseline: 1.0476x; 1.0476x over previous
import jax
import jax.numpy as jnp
from jax import lax
from jax.experimental import pallas as pl
from jax.experimental.pallas import tpu as pltpu
from jax.experimental.pallas import tpu_sc as plsc

N_NODES = 10000
D = 128
E = 320000
N_PAD = 10240
NC, NS, L = 2, 16, 16
NW = NC * NS
E_PER_W = E // NW
CH = 128
E_PAD_PER_W = 10240
NCHUNK = E_PAD_PER_W // CH
DEG_ROWS = N_PAD // D
ACC_ROWS_PER_TILE = N_PAD // NS
WB_CH = 128
RB = 1024

_mesh = plsc.VectorSubcoreMesh(core_axis_name="c", subcore_axis_name="s")
_sc_params = pltpu.CompilerParams(needs_layout_passes=False)


def _deg_body(dst_hbm, out_hbm, idx_v, deg_v):
    c = lax.axis_index("c")
    s = lax.axis_index("s")
    wid = c * NS + s

    pltpu.sync_copy(dst_hbm.at[wid], idx_v)

    zeros16 = jnp.zeros((L,), jnp.float32)

    def zero_row(i, carry):
        deg_v[pl.ds(i * L, L)] = zeros16
        return carry

    lax.fori_loop(0, N_PAD // L, zero_row, 0)

    ones16 = jnp.full((L,), 1.0, jnp.float32)

    def body(i, carry):
        for u in range(4):
            vec = idx_v[pl.ds((i * 4 + u) * L, L)]
            plsc.addupdate_scatter(deg_v, [vec], ones16)
        return carry

    lax.fori_loop(0, E_PER_W // (4 * L), body, 0)

    pltpu.sync_copy(deg_v, out_hbm.at[wid])


_deg_call = pl.kernel(
    _deg_body,
    out_type=jax.ShapeDtypeStruct((NW, N_PAD), jnp.float32),
    mesh=_mesh,
    scratch_types=[
        pltpu.VMEM((E_PER_W,), jnp.int32),
        pltpu.VMEM((N_PAD,), jnp.float32),
    ],
    compiler_params=_sc_params,
)


def _prep_body(x_ref, w_ref, degp_ref, g_ref, dinv_ref):
    d = jnp.sum(degp_ref[...], axis=0) + 1.0
    dinv = lax.rsqrt(d)
    dinv_ref[...] = dinv
    xs = x_ref[...] * dinv[:, None]
    g_ref[...] = lax.dot_general(
        xs, w_ref[...], (((1,), (1,)), ((), ())),
        preferred_element_type=jnp.float32)


_prep_call = pl.pallas_call(
    _prep_body,
    grid=(N_PAD // RB,),
    in_specs=[
        pl.BlockSpec((RB, D), lambda i: (i, 0)),
        pl.BlockSpec((D, D), lambda i: (0, 0)),
        pl.BlockSpec((NW, RB), lambda i: (0, i)),
    ],
    out_specs=[
        pl.BlockSpec((RB, D), lambda i: (i, 0)),
        pl.BlockSpec((RB,), lambda i: (i,)),
    ],
    out_shape=[
        jax.ShapeDtypeStruct((N_PAD, D), jnp.float32),
        jax.ShapeDtypeStruct((N_PAD,), jnp.float32),
    ],
)


HNCHUNK = NCHUNK // 2


def _edge_body(src_hbm, dst_hbm, g_hbm, out_hbm, src_v, dst_v, rows0_v,
               rows1_v, acc_sh, semg0, semg1):
    c = lax.axis_index("c")
    s = lax.axis_index("s")
    wid = c * NS + s

    zeros16 = jnp.zeros((L,), jnp.float32)

    def zero_row(i, carry):
        for j in range(D // L):
            rows0_v[i, pl.ds(j * L, L)] = zeros16
        return carry

    lax.fori_loop(0, WB_CH, zero_row, 0)

    base = s * ACC_ROWS_PER_TILE
    zsrc = rows0_v.at[pl.ds(0, WB_CH)]

    def zero_acc(k, carry):
        pltpu.sync_copy(zsrc, acc_sh.at[pl.ds(base + k * WB_CH, WB_CH)])
        return carry

    lax.fori_loop(0, ACC_ROWS_PER_TILE // WB_CH, zero_acc, 0)

    plsc.subcore_barrier()

    for h in range(NCHUNK // HNCHUNK):
        pltpu.sync_copy(src_hbm.at[wid].at[pl.ds(h * HNCHUNK, HNCHUNK)],
                        src_v)
        pltpu.sync_copy(dst_hbm.at[wid].at[pl.ds(h * HNCHUNK, HNCHUNK)],
                        dst_v)
        pltpu.async_copy(g_hbm.at[src_v.at[0]], rows0_v, semg0)

        def body(i, carry):
            j = i * 2
            pltpu.async_copy(g_hbm.at[src_v.at[j + 1]], rows1_v, semg1)
            pltpu.make_async_copy(g_hbm.at[src_v.at[j]], rows0_v,
                                  semg0).wait()
            pltpu.sync_copy(rows0_v, acc_sh.at[dst_v.at[j]], add=True)

            @pl.when(j + 2 < HNCHUNK)
            def _():
                pltpu.async_copy(g_hbm.at[src_v.at[j + 2]], rows0_v, semg0)

            pltpu.make_async_copy(g_hbm.at[src_v.at[j + 1]], rows1_v,
                                  semg1).wait()
            pltpu.sync_copy(rows1_v, acc_sh.at[dst_v.at[j + 1]], add=True)
            return carry

        lax.fori_loop(0, HNCHUNK // 2, body, 0)

    plsc.subcore_barrier()

    wb = rows0_v.at[pl.ds(0, WB_CH)]

    def writeback(k, carry):
        r = base + k * WB_CH
        pltpu.sync_copy(acc_sh.at[pl.ds(r, WB_CH)], wb)
        pltpu.sync_copy(wb, out_hbm.at[c].at[pl.ds(r, WB_CH)])
        return carry

    lax.fori_loop(0, ACC_ROWS_PER_TILE // WB_CH, writeback, 0)


_edge_call = pl.kernel(
    _edge_body,
    out_type=jax.ShapeDtypeStruct((NC, N_PAD, D), jnp.float32),
    mesh=_mesh,
    scratch_types=[
        pltpu.VMEM((HNCHUNK, CH), jnp.int32),
        pltpu.VMEM((HNCHUNK, CH), jnp.int32),
        pltpu.VMEM((CH, D), jnp.float32),
        pltpu.VMEM((CH, D), jnp.float32),
        pltpu.VMEM_SHARED((N_PAD, D), jnp.float32),
        pltpu.SemaphoreType.DMA,
        pltpu.SemaphoreType.DMA,
    ],
    compiler_params=_sc_params,
)


def _fin_body(p_ref, g_ref, dinv_ref, b_ref, o_ref):
    acc = p_ref[0] + p_ref[1] + g_ref[...]
    o_ref[...] = jnp.maximum(acc * dinv_ref[...][:, None] + b_ref[...], 0.0)


_fin_call = pl.pallas_call(
    _fin_body,
    grid=(N_PAD // RB,),
    in_specs=[
        pl.BlockSpec((NC, RB, D), lambda i: (0, i, 0)),
        pl.BlockSpec((RB, D), lambda i: (i, 0)),
        pl.BlockSpec((RB,), lambda i: (i,)),
        pl.BlockSpec((1, D), lambda i: (0, 0)),
    ],
    out_specs=pl.BlockSpec((RB, D), lambda i: (i, 0)),
    out_shape=jax.ShapeDtypeStruct((N_PAD, D), jnp.float32),
)


def kernel(x, edge_index, pos, batch, W, b):
    src = edge_index[0].astype(jnp.int32)
    dst = edge_index[1].astype(jnp.int32)
    pad_w = E_PAD_PER_W - E_PER_W
    pad_ids = (jnp.arange(pad_w, dtype=jnp.int32) % (N_PAD - N_NODES)
               ) + N_NODES
    pad_blk = jnp.broadcast_to(pad_ids, (NW, pad_w))
    src_p = jnp.concatenate([src.reshape(NW, E_PER_W), pad_blk],
                            axis=1).reshape(NW, NCHUNK, CH)
    dst_p = jnp.concatenate([dst.reshape(NW, E_PER_W), pad_blk],
                            axis=1).reshape(NW, NCHUNK, CH)
    dst_flat = dst.reshape(NW, E_PER_W)

    xp = jnp.pad(x, ((0, N_PAD - N_NODES), (0, 0)))

    degp = _deg_call(dst_flat)
    g, dinv = _prep_call(xp, W, degp)
    p = _edge_call(src_p, dst_p, g)
    out_pad = _fin_call(p, g, dinv, b.reshape(1, D))
    return (out_pad[:N_NODES], edge_index, pos)
